# baseline (speedup 1.0000x reference)
import jax
import jax.numpy as jnp
from jax.experimental import pallas as pl
from jax.experimental.pallas import tpu as pltpu

_N_CHAINS = 8


def _copy_body(table_ref, out_ref, vmem, *sems):
    rows = out_ref.shape[0]
    chunk = rows // _N_CHAINS
    lsems = sems[:_N_CHAINS]
    ssems = sems[_N_CHAINS:]
    loads = [
        pltpu.make_async_copy(
            table_ref.at[pl.ds(i * chunk, chunk)], vmem.at[i], lsems[i]
        )
        for i in range(_N_CHAINS)
    ]
    stores = [
        pltpu.make_async_copy(
            vmem.at[i], out_ref.at[pl.ds(i * chunk, chunk)], ssems[i]
        )
        for i in range(_N_CHAINS)
    ]
    for c in loads:
        c.start()
    for i in range(_N_CHAINS):
        loads[i].wait()
        stores[i].start()
    for c in stores:
        c.wait()


def kernel(x, table):
    seq_len = x.shape[-1]
    hidden = table.shape[1]
    chunk = seq_len // _N_CHAINS
    return pl.pallas_call(
        _copy_body,
        in_specs=[pl.BlockSpec(memory_space=pl.ANY)],
        out_specs=pl.BlockSpec(memory_space=pl.ANY),
        out_shape=jax.ShapeDtypeStruct((seq_len, hidden), table.dtype),
        scratch_shapes=[pltpu.VMEM((_N_CHAINS, chunk, hidden), table.dtype)]
        + [pltpu.SemaphoreType.DMA] * (2 * _N_CHAINS),
    )(table)
